# Initial kernel scaffold; baseline (speedup 1.0000x reference)
import functools

import jax
import jax.numpy as jnp
from jax import lax
from jax.experimental import pallas as pl
from jax.experimental.pallas import tpu as pltpu
from jax.experimental.pallas import tpu_sc as plsc

N = 50000
M = 16
ORIG = 92
NBR = 41
D = 64
H = 128
NCONV = 3
NCRYS = 512
EPS = 1e-5
NM = N * M

NW = 32
CH = 128
CPW = (NM + NW * CH - 1) // (NW * CH)
NMP = NW * CPW * CH

BN1 = 400
EB = BN1 * M
NB1 = N // BN1
BN3 = 2000
NB3 = N // BN3
BP = 2000
NBP = N // BP


def _sc_gather(table, idx2d):
    mesh = plsc.VectorSubcoreMesh(core_axis_name="c", subcore_axis_name="s")

    @functools.partial(
        pl.kernel,
        out_type=jax.ShapeDtypeStruct((NMP, D), jnp.float32),
        mesh=mesh,
        scratch_types=[
            pltpu.VMEM((CPW, CH), jnp.int32),
            pltpu.VMEM((2, CH, D), jnp.float32),
            pltpu.SemaphoreType.DMA,
            pltpu.SemaphoreType.DMA,
            pltpu.SemaphoreType.DMA,
        ],
    )
    def k(table_hbm, idx_hbm, out_hbm, idx_v, bufs, gsem, ssem0, ssem1):
        wid = lax.axis_index("s") * 2 + lax.axis_index("c")
        row0 = wid * CPW
        pltpu.sync_copy(idx_hbm.at[pl.ds(row0, CPW)], idx_v)

        pltpu.async_copy(table_hbm.at[idx_v.at[0]], bufs.at[0], gsem)

        def body(j, _):
            slot = lax.rem(j, 2)
            nxt = 1 - slot

            @pl.when(j + 1 < CPW)
            def _():
                pltpu.async_copy(table_hbm.at[idx_v.at[j + 1]], bufs.at[nxt], gsem)

            pltpu.make_async_copy(table_hbm.at[idx_v.at[j]], bufs.at[slot], gsem).wait()
            dst = out_hbm.at[pl.ds((row0 + j) * CH, CH)]

            @pl.when(slot == 0)
            def _():
                pltpu.async_copy(bufs.at[0], dst, ssem0).wait()

            @pl.when(slot == 1)
            def _():
                pltpu.async_copy(bufs.at[1], dst, ssem1).wait()

            return 0

        lax.fori_loop(0, CPW, body, 0)

    return k(table, idx2d)


def _embed_body(x_ref, w_ref, b_ref, o_ref):
    o_ref[...] = (
        jnp.dot(x_ref[...], w_ref[...], preferred_element_type=jnp.float32)
        + b_ref[...]
    )


def _embed(atom_fea, emb_W, emb_b):
    return pl.pallas_call(
        _embed_body,
        grid=(NB3,),
        in_specs=[
            pl.BlockSpec((BN3, ORIG), lambda i: (i, 0)),
            pl.BlockSpec((ORIG, D), lambda i: (0, 0)),
            pl.BlockSpec((1, D), lambda i: (0, 0)),
        ],
        out_specs=pl.BlockSpec((BN3, D), lambda i: (i, 0)),
        out_shape=jax.ShapeDtypeStruct((N, D), jnp.float32),
    )(atom_fea, emb_W, emb_b.reshape(1, D))


def _z_halves(atom_ref, anbr_ref, fea_ref, wsf, wsc, wnf, wnc, wff, wfc):
    a = atom_ref[...]
    nb = anbr_ref[...]
    fe = fea_ref[...]
    f32 = jnp.float32
    sf = jnp.dot(a, wsf[...], preferred_element_type=f32)
    sc = jnp.dot(a, wsc[...], preferred_element_type=f32)
    nf = jnp.dot(nb, wnf[...], preferred_element_type=f32)
    nc = jnp.dot(nb, wnc[...], preferred_element_type=f32)
    ff = jnp.dot(fe, wff[...], preferred_element_type=f32)
    fc = jnp.dot(fe, wfc[...], preferred_element_type=f32)
    zf = nf.reshape(BN1, M, D) + ff.reshape(BN1, M, D) + sf[:, None, :]
    zc = nc.reshape(BN1, M, D) + fc.reshape(BN1, M, D) + sc[:, None, :]
    return zf, zc


def _p1_body(atom_ref, anbr_ref, fea_ref, wsf, wsc, wnf, wnc, wff, wfc,
             outf_ref, outc_ref, accf, accc):
    i = pl.program_id(0)

    @pl.when(i == 0)
    def _():
        accf[...] = jnp.zeros_like(accf)
        accc[...] = jnp.zeros_like(accc)

    zf, zc = _z_halves(atom_ref, anbr_ref, fea_ref, wsf, wsc, wnf, wnc, wff, wfc)
    accf[0, :] += jnp.sum(zf, axis=(0, 1))
    accf[1, :] += jnp.sum(zf * zf, axis=(0, 1))
    accc[0, :] += jnp.sum(zc, axis=(0, 1))
    accc[1, :] += jnp.sum(zc * zc, axis=(0, 1))

    @pl.when(i == NB1 - 1)
    def _():
        for acc, out in ((accf, outf_ref), (accc, outc_ref)):
            mu = acc[0, :] / NM
            var = acc[1, :] / NM - mu * mu
            out[0, :] = mu
            out[1, :] = lax.rsqrt(var + EPS)


def _pass1(atom, anbr, fea2, ws):
    return pl.pallas_call(
        _p1_body,
        grid=(NB1,),
        in_specs=[
            pl.BlockSpec((BN1, D), lambda i: (i, 0)),
            pl.BlockSpec((EB, D), lambda i: (i, 0)),
            pl.BlockSpec((EB, NBR), lambda i: (i, 0)),
        ] + [pl.BlockSpec(w.shape, lambda i: (0, 0)) for w in ws],
        out_specs=[
            pl.BlockSpec((2, D), lambda i: (0, 0)),
            pl.BlockSpec((2, D), lambda i: (0, 0)),
        ],
        out_shape=[
            jax.ShapeDtypeStruct((2, D), jnp.float32),
            jax.ShapeDtypeStruct((2, D), jnp.float32),
        ],
        scratch_shapes=[
            pltpu.VMEM((2, D), jnp.float32),
            pltpu.VMEM((2, D), jnp.float32),
        ],
    )(atom, anbr, fea2, *ws)


def _p2_body(atom_ref, anbr_ref, fea_ref, wsf, wsc, wnf, wnc, wff, wfc,
             stf, stc, g1f, b1f, g1c, b1c,
             sum_ref, out2_ref, acc2):
    i = pl.program_id(0)

    @pl.when(i == 0)
    def _():
        acc2[...] = jnp.zeros_like(acc2)

    zf, zc = _z_halves(atom_ref, anbr_ref, fea_ref, wsf, wsc, wnf, wnc, wff, wfc)
    nzf = (zf - stf[0, :][None, None, :]) * stf[1, :][None, None, :] \
        * g1f[0, :][None, None, :] + b1f[0, :][None, None, :]
    nzc = (zc - stc[0, :][None, None, :]) * stc[1, :][None, None, :] \
        * g1c[0, :][None, None, :] + b1c[0, :][None, None, :]
    filt = jax.nn.sigmoid(nzf)
    core = jax.nn.softplus(nzc)
    s = jnp.sum(filt * core, axis=1)
    sum_ref[...] = s
    acc2[0, :] += jnp.sum(s, axis=0)
    acc2[1, :] += jnp.sum(s * s, axis=0)

    @pl.when(i == NB1 - 1)
    def _():
        mu = acc2[0, :] / N
        var = acc2[1, :] / N - mu * mu
        out2_ref[0, :] = mu
        out2_ref[1, :] = lax.rsqrt(var + EPS)


def _pass2(atom, anbr, fea2, ws, stf, stc, g1f, b1f, g1c, b1c):
    return pl.pallas_call(
        _p2_body,
        grid=(NB1,),
        in_specs=[
            pl.BlockSpec((BN1, D), lambda i: (i, 0)),
            pl.BlockSpec((EB, D), lambda i: (i, 0)),
            pl.BlockSpec((EB, NBR), lambda i: (i, 0)),
        ] + [pl.BlockSpec(w.shape, lambda i: (0, 0)) for w in ws]
          + [pl.BlockSpec((2, D), lambda i: (0, 0))] * 2
          + [pl.BlockSpec((1, D), lambda i: (0, 0))] * 4,
        out_specs=[
            pl.BlockSpec((BN1, D), lambda i: (i, 0)),
            pl.BlockSpec((2, D), lambda i: (0, 0)),
        ],
        out_shape=[
            jax.ShapeDtypeStruct((N, D), jnp.float32),
            jax.ShapeDtypeStruct((2, D), jnp.float32),
        ],
        scratch_shapes=[pltpu.VMEM((2, D), jnp.float32)],
    )(atom, anbr, fea2, *ws, stf, stc, g1f, b1f, g1c, b1c)


def _p3_body(atom_ref, sum_ref, st2, g2, b2, o_ref):
    s = (sum_ref[...] - st2[0, :][None, :]) * st2[1, :][None, :] \
        * g2[0, :][None, :] + b2[0, :][None, :]
    o_ref[...] = jax.nn.softplus(atom_ref[...] + s)


def _pass3(atom, summed, st2, g2, b2):
    return pl.pallas_call(
        _p3_body,
        grid=(NB3,),
        in_specs=[
            pl.BlockSpec((BN3, D), lambda i: (i, 0)),
            pl.BlockSpec((BN3, D), lambda i: (i, 0)),
            pl.BlockSpec((2, D), lambda i: (0, 0)),
            pl.BlockSpec((1, D), lambda i: (0, 0)),
            pl.BlockSpec((1, D), lambda i: (0, 0)),
        ],
        out_specs=pl.BlockSpec((BN3, D), lambda i: (i, 0)),
        out_shape=jax.ShapeDtypeStruct((N, D), jnp.float32),
    )(atom, summed, st2, g2, b2)


def _pool_body(atom_ref, ids_ref, cfw, cfb, outw, outb, o_ref, acc_s, acc_c):
    i = pl.program_id(0)

    @pl.when(i == 0)
    def _():
        acc_s[...] = jnp.zeros_like(acc_s)
        acc_c[...] = jnp.zeros_like(acc_c)

    ids = ids_ref[0, 0, :]
    oh = (lax.broadcasted_iota(jnp.int32, (NCRYS, BP), 0)
          == ids[None, :]).astype(jnp.float32)
    acc_s[...] += jnp.dot(oh, atom_ref[...], preferred_element_type=jnp.float32)
    acc_c[...] += jnp.sum(oh, axis=1, keepdims=True)

    @pl.when(i == NBP - 1)
    def _():
        mean = acc_s[...] / jnp.maximum(acc_c[...], 1.0)
        t = jax.nn.softplus(mean)
        h = jax.nn.softplus(
            jnp.dot(t, cfw[...], preferred_element_type=jnp.float32) + cfb[...]
        )
        o_ref[...] = jnp.sum(h * outw[...], axis=1, keepdims=True) + outb[...]


def _pool(atom, ids3d, cf_W, cf_b, out_W, out_b):
    return pl.pallas_call(
        _pool_body,
        grid=(NBP,),
        in_specs=[
            pl.BlockSpec((BP, D), lambda i: (i, 0)),
            pl.BlockSpec((1, 1, BP), lambda i: (i, 0, 0)),
            pl.BlockSpec((D, H), lambda i: (0, 0)),
            pl.BlockSpec((1, H), lambda i: (0, 0)),
            pl.BlockSpec((1, H), lambda i: (0, 0)),
            pl.BlockSpec((1, 1), lambda i: (0, 0)),
        ],
        out_specs=pl.BlockSpec((NCRYS, 1), lambda i: (0, 0)),
        out_shape=jax.ShapeDtypeStruct((NCRYS, 1), jnp.float32),
        scratch_shapes=[
            pltpu.VMEM((NCRYS, D), jnp.float32),
            pltpu.VMEM((NCRYS, 1), jnp.float32),
        ],
    )(atom, ids3d, cf_W, cf_b, out_W, out_b)


def kernel(atom_fea, nbr_fea, nbr_fea_idx, crystal_ids, emb_W, emb_b,
           fc_W, fc_b, bn1_g, bn1_b, bn2_g, bn2_b, cf_W, cf_b, out_W, out_b):
    idx_flat = nbr_fea_idx.reshape(NM).astype(jnp.int32)
    idx2d = jnp.pad(idx_flat, (0, NMP - NM)).reshape(NW * CPW, CH)
    fea2 = nbr_fea.reshape(NM, NBR)
    ids3d = crystal_ids.astype(jnp.int32).reshape(NBP, 1, BP)

    atom = _embed(atom_fea, emb_W, emb_b)

    for i in range(NCONV):
        w = fc_W[i]
        ws = (w[:D, :D], w[:D, D:], w[D:2 * D, :D], w[D:2 * D, D:],
              w[2 * D:, :D], w[2 * D:, D:])
        g1f = bn1_g[i][:D].reshape(1, D)
        g1c = bn1_g[i][D:].reshape(1, D)
        b1f = bn1_b[i][:D].reshape(1, D)
        b1c = bn1_b[i][D:].reshape(1, D)
        g2 = bn2_g[i].reshape(1, D)
        b2 = bn2_b[i].reshape(1, D)

        anbr = _sc_gather(atom, idx2d)
        stf, stc = _pass1(atom, anbr, fea2, ws)
        summed, st2 = _pass2(atom, anbr, fea2, ws, stf, stc, g1f, b1f, g1c, b1c)
        atom = _pass3(atom, summed, st2, g2, b2)

    return _pool(atom, ids3d, cf_W, cf_b, out_W.reshape(1, H), out_b.reshape(1, 1))


# speedup vs baseline: 1.9867x; 1.9867x over previous
import functools

import jax
import jax.numpy as jnp
from jax import lax
from jax.experimental import pallas as pl
from jax.experimental.pallas import tpu as pltpu
from jax.experimental.pallas import tpu_sc as plsc

N = 50000
M = 16
ORIG = 92
NBR = 41
D = 64
D2 = 128
H = 128
NCONV = 3
NCRYS = 512
EPS = 1e-5
NM = N * M

NW = 32
CH = 128
CPW = (NM + NW * CH - 1) // (NW * CH)
NMP = NW * CPW * CH

BN1 = 400
EB = BN1 * M
NB1 = N // BN1
BN3 = 2000
NB3 = N // BN3
BP = 2000
NBP = N // BP


def _sc_gather(table, idx3d):
    mesh = plsc.VectorSubcoreMesh(core_axis_name="c", subcore_axis_name="s")

    @functools.partial(
        pl.kernel,
        out_type=jax.ShapeDtypeStruct((NMP, D2), jnp.float32),
        mesh=mesh,
        scratch_types=[
            pltpu.VMEM((CPW, CH), jnp.int32),
            pltpu.VMEM((2, CH, D2), jnp.float32),
            pltpu.SemaphoreType.DMA,
            pltpu.SemaphoreType.DMA,
            pltpu.SemaphoreType.DMA,
        ],
    )
    def k(table_hbm, idx_hbm, out_hbm, idx_v, bufs, gsem, ssem0, ssem1):
        wid = lax.axis_index("s") * 2 + lax.axis_index("c")
        row0 = wid * CPW
        pltpu.sync_copy(idx_hbm.at[wid], idx_v)

        pltpu.async_copy(table_hbm.at[idx_v.at[0]], bufs.at[0], gsem)

        def body(j, _):
            slot = lax.rem(j, 2)
            nxt = 1 - slot

            @pl.when(j + 1 < CPW)
            def _():
                pltpu.async_copy(table_hbm.at[idx_v.at[j + 1]], bufs.at[nxt], gsem)

            pltpu.make_async_copy(table_hbm.at[idx_v.at[j]], bufs.at[slot], gsem).wait()
            dst = out_hbm.at[pl.ds((row0 + j) * CH, CH)]

            @pl.when(slot == 0)
            def _():
                pltpu.async_copy(bufs.at[0], dst, ssem0).wait()

            @pl.when(slot == 1)
            def _():
                pltpu.async_copy(bufs.at[1], dst, ssem1).wait()

            return 0

        lax.fori_loop(0, CPW, body, 0)

    return k(table, idx3d)


def _embed_body(x_ref, w_ref, b_ref, o_ref):
    o_ref[...] = (
        jnp.dot(x_ref[...], w_ref[...], preferred_element_type=jnp.float32)
        + b_ref[...]
    )


def _embed(atom_fea, emb_Wp, emb_bp):
    return pl.pallas_call(
        _embed_body,
        grid=(NB3,),
        in_specs=[
            pl.BlockSpec((BN3, ORIG), lambda i: (i, 0)),
            pl.BlockSpec((ORIG, D2), lambda i: (0, 0)),
            pl.BlockSpec((1, D2), lambda i: (0, 0)),
        ],
        out_specs=pl.BlockSpec((BN3, D2), lambda i: (i, 0)),
        out_shape=jax.ShapeDtypeStruct((N, D2), jnp.float32),
    )(atom_fea, emb_Wp, emb_bp)


def _z_halves(atom_ref, anbr_ref, fea_ref, wsf, wsc, wnf, wnc, wff, wfc):
    a = atom_ref[...]
    nb = anbr_ref[...]
    fe = fea_ref[...]
    f32 = jnp.float32
    sf = jnp.dot(a, wsf[...], preferred_element_type=f32)
    sc = jnp.dot(a, wsc[...], preferred_element_type=f32)
    nf = jnp.dot(nb, wnf[...], preferred_element_type=f32)
    nc = jnp.dot(nb, wnc[...], preferred_element_type=f32)
    ff = jnp.dot(fe, wff[...], preferred_element_type=f32)
    fc = jnp.dot(fe, wfc[...], preferred_element_type=f32)
    zf = nf.reshape(BN1, M, D) + ff.reshape(BN1, M, D) + sf[:, None, :]
    zc = nc.reshape(BN1, M, D) + fc.reshape(BN1, M, D) + sc[:, None, :]
    return zf, zc


def _p1_body(atom_ref, anbr_ref, fea_ref, wsf, wsc, wnf, wnc, wff, wfc,
             outf_ref, outc_ref, accf, accc):
    i = pl.program_id(0)

    @pl.when(i == 0)
    def _():
        accf[...] = jnp.zeros_like(accf)
        accc[...] = jnp.zeros_like(accc)

    zf, zc = _z_halves(atom_ref, anbr_ref, fea_ref, wsf, wsc, wnf, wnc, wff, wfc)
    accf[0, :] += jnp.sum(zf, axis=(0, 1))
    accf[1, :] += jnp.sum(zf * zf, axis=(0, 1))
    accc[0, :] += jnp.sum(zc, axis=(0, 1))
    accc[1, :] += jnp.sum(zc * zc, axis=(0, 1))

    @pl.when(i == NB1 - 1)
    def _():
        for acc, out in ((accf, outf_ref), (accc, outc_ref)):
            mu = acc[0, :] / NM
            var = acc[1, :] / NM - mu * mu
            out[0, :] = mu
            out[1, :] = lax.rsqrt(var + EPS)


def _pass1(atom, anbr, fea2, ws):
    return pl.pallas_call(
        _p1_body,
        grid=(NB1,),
        in_specs=[
            pl.BlockSpec((BN1, D2), lambda i: (i, 0)),
            pl.BlockSpec((EB, D2), lambda i: (i, 0)),
            pl.BlockSpec((EB, NBR), lambda i: (i, 0)),
        ] + [pl.BlockSpec(w.shape, lambda i: (0, 0)) for w in ws],
        out_specs=[
            pl.BlockSpec((2, D), lambda i: (0, 0)),
            pl.BlockSpec((2, D), lambda i: (0, 0)),
        ],
        out_shape=[
            jax.ShapeDtypeStruct((2, D), jnp.float32),
            jax.ShapeDtypeStruct((2, D), jnp.float32),
        ],
        scratch_shapes=[
            pltpu.VMEM((2, D), jnp.float32),
            pltpu.VMEM((2, D), jnp.float32),
        ],
    )(atom, anbr, fea2, *ws)


def _p2_body(atom_ref, anbr_ref, fea_ref, wsf, wsc, wnf, wnc, wff, wfc,
             stf, stc, g1f, b1f, g1c, b1c,
             sum_ref, out2_ref, acc2):
    i = pl.program_id(0)

    @pl.when(i == 0)
    def _():
        acc2[...] = jnp.zeros_like(acc2)

    zf, zc = _z_halves(atom_ref, anbr_ref, fea_ref, wsf, wsc, wnf, wnc, wff, wfc)
    nzf = (zf - stf[0, :][None, None, :]) * stf[1, :][None, None, :] \
        * g1f[0, :][None, None, :] + b1f[0, :][None, None, :]
    nzc = (zc - stc[0, :][None, None, :]) * stc[1, :][None, None, :] \
        * g1c[0, :][None, None, :] + b1c[0, :][None, None, :]
    filt = jax.nn.sigmoid(nzf)
    core = jax.nn.softplus(nzc)
    s = jnp.sum(filt * core, axis=1)
    sum_ref[...] = s
    acc2[0, :] += jnp.sum(s, axis=0)
    acc2[1, :] += jnp.sum(s * s, axis=0)

    @pl.when(i == NB1 - 1)
    def _():
        mu = acc2[0, :] / N
        var = acc2[1, :] / N - mu * mu
        out2_ref[0, :] = mu
        out2_ref[1, :] = lax.rsqrt(var + EPS)


def _pass2(atom, anbr, fea2, ws, stf, stc, g1f, b1f, g1c, b1c):
    return pl.pallas_call(
        _p2_body,
        grid=(NB1,),
        in_specs=[
            pl.BlockSpec((BN1, D2), lambda i: (i, 0)),
            pl.BlockSpec((EB, D2), lambda i: (i, 0)),
            pl.BlockSpec((EB, NBR), lambda i: (i, 0)),
        ] + [pl.BlockSpec(w.shape, lambda i: (0, 0)) for w in ws]
          + [pl.BlockSpec((2, D), lambda i: (0, 0))] * 2
          + [pl.BlockSpec((1, D), lambda i: (0, 0))] * 4,
        out_specs=[
            pl.BlockSpec((BN1, D), lambda i: (i, 0)),
            pl.BlockSpec((2, D), lambda i: (0, 0)),
        ],
        out_shape=[
            jax.ShapeDtypeStruct((N, D), jnp.float32),
            jax.ShapeDtypeStruct((2, D), jnp.float32),
        ],
        scratch_shapes=[pltpu.VMEM((2, D), jnp.float32)],
    )(atom, anbr, fea2, *ws, stf, stc, g1f, b1f, g1c, b1c)


def _p3_body(atom_ref, sum_ref, st2, g2, b2, o_ref):
    s = (sum_ref[...] - st2[0, :][None, :]) * st2[1, :][None, :] \
        * g2[0, :][None, :] + b2[0, :][None, :]
    sp = jnp.concatenate([s, jnp.zeros((BN3, D2 - D), jnp.float32)], axis=1)
    o_ref[...] = jax.nn.softplus(atom_ref[...] + sp)


def _pass3(atom, summed, st2, g2, b2):
    return pl.pallas_call(
        _p3_body,
        grid=(NB3,),
        in_specs=[
            pl.BlockSpec((BN3, D2), lambda i: (i, 0)),
            pl.BlockSpec((BN3, D), lambda i: (i, 0)),
            pl.BlockSpec((2, D), lambda i: (0, 0)),
            pl.BlockSpec((1, D), lambda i: (0, 0)),
            pl.BlockSpec((1, D), lambda i: (0, 0)),
        ],
        out_specs=pl.BlockSpec((BN3, D2), lambda i: (i, 0)),
        out_shape=jax.ShapeDtypeStruct((N, D2), jnp.float32),
    )(atom, summed, st2, g2, b2)


def _pool_body(atom_ref, ids_ref, cfw, cfb, outw, outb, o_ref, acc_s, acc_c):
    i = pl.program_id(0)

    @pl.when(i == 0)
    def _():
        acc_s[...] = jnp.zeros_like(acc_s)
        acc_c[...] = jnp.zeros_like(acc_c)

    ids = ids_ref[0, 0, :]
    oh = (lax.broadcasted_iota(jnp.int32, (NCRYS, BP), 0)
          == ids[None, :]).astype(jnp.float32)
    acc_s[...] += jnp.dot(oh, atom_ref[...], preferred_element_type=jnp.float32)
    acc_c[...] += jnp.sum(oh, axis=1, keepdims=True)

    @pl.when(i == NBP - 1)
    def _():
        mean = acc_s[...] / jnp.maximum(acc_c[...], 1.0)
        t = jax.nn.softplus(mean)
        h = jax.nn.softplus(
            jnp.dot(t, cfw[...], preferred_element_type=jnp.float32) + cfb[...]
        )
        o_ref[...] = jnp.sum(h * outw[...], axis=1, keepdims=True) + outb[...]


def _pool(atom, ids3d, cf_Wp, cf_b, out_W, out_b):
    return pl.pallas_call(
        _pool_body,
        grid=(NBP,),
        in_specs=[
            pl.BlockSpec((BP, D2), lambda i: (i, 0)),
            pl.BlockSpec((1, 1, BP), lambda i: (i, 0, 0)),
            pl.BlockSpec((D2, H), lambda i: (0, 0)),
            pl.BlockSpec((1, H), lambda i: (0, 0)),
            pl.BlockSpec((1, H), lambda i: (0, 0)),
            pl.BlockSpec((1, 1), lambda i: (0, 0)),
        ],
        out_specs=pl.BlockSpec((NCRYS, 1), lambda i: (0, 0)),
        out_shape=jax.ShapeDtypeStruct((NCRYS, 1), jnp.float32),
        scratch_shapes=[
            pltpu.VMEM((NCRYS, D2), jnp.float32),
            pltpu.VMEM((NCRYS, 1), jnp.float32),
        ],
    )(atom, ids3d, cf_Wp, cf_b, out_W, out_b)


def _rpad(a, rows):
    return jnp.pad(a, ((0, rows - a.shape[0]), (0, 0)))


def kernel(atom_fea, nbr_fea, nbr_fea_idx, crystal_ids, emb_W, emb_b,
           fc_W, fc_b, bn1_g, bn1_b, bn2_g, bn2_b, cf_W, cf_b, out_W, out_b):
    idx_flat = nbr_fea_idx.reshape(NM).astype(jnp.int32)
    idx3d = jnp.pad(idx_flat, (0, NMP - NM)).reshape(NW, CPW, CH)
    fea2 = nbr_fea.reshape(NM, NBR)
    ids3d = crystal_ids.astype(jnp.int32).reshape(NBP, 1, BP)
    emb_Wp = jnp.pad(emb_W, ((0, 0), (0, D2 - D)))
    emb_bp = jnp.pad(emb_b, (0, D2 - D)).reshape(1, D2)
    cf_Wp = _rpad(cf_W, D2)

    atom = _embed(atom_fea, emb_Wp, emb_bp)

    for i in range(NCONV):
        w = fc_W[i]
        ws = (_rpad(w[:D, :D], D2), _rpad(w[:D, D:], D2),
              _rpad(w[D:2 * D, :D], D2), _rpad(w[D:2 * D, D:], D2),
              w[2 * D:, :D], w[2 * D:, D:])
        g1f = bn1_g[i][:D].reshape(1, D)
        g1c = bn1_g[i][D:].reshape(1, D)
        b1f = bn1_b[i][:D].reshape(1, D)
        b1c = bn1_b[i][D:].reshape(1, D)
        g2 = bn2_g[i].reshape(1, D)
        b2 = bn2_b[i].reshape(1, D)

        anbr = _sc_gather(atom, idx3d)
        stf, stc = _pass1(atom, anbr, fea2, ws)
        summed, st2 = _pass2(atom, anbr, fea2, ws, stf, stc, g1f, b1f, g1c, b1c)
        atom = _pass3(atom, summed, st2, g2, b2)

    return _pool(atom, ids3d, cf_Wp, cf_b.reshape(1, H), out_W.reshape(1, H),
                 out_b.reshape(1, 1))


# speedup vs baseline: 2.2362x; 1.1256x over previous
import functools

import jax
import jax.numpy as jnp
from jax import lax
from jax.experimental import pallas as pl
from jax.experimental.pallas import tpu as pltpu
from jax.experimental.pallas import tpu_sc as plsc

N = 50000
M = 16
ORIG = 92
NBR = 41
D = 64
D2 = 128
H = 128
NCONV = 3
NCRYS = 512
EPS = 1e-5
NM = N * M

NW = 32
CH = 128
CPW = (NM + NW * CH - 1) // (NW * CH)
NMP = NW * CPW * CH

BN1 = 400
EB = BN1 * M
NB1 = N // BN1
BN3 = 2000
NB3 = N // BN3
BP = 2000
NBP = N // BP


def _sc_gather(table, idx3d):
    mesh = plsc.VectorSubcoreMesh(core_axis_name="c", subcore_axis_name="s")

    @functools.partial(
        pl.kernel,
        out_type=jax.ShapeDtypeStruct((NMP, D2), jnp.float32),
        mesh=mesh,
        scratch_types=[
            pltpu.VMEM((CPW, CH), jnp.int32),
            pltpu.VMEM((2, CH, D2), jnp.float32),
            pltpu.SemaphoreType.DMA,
            pltpu.SemaphoreType.DMA,
            pltpu.SemaphoreType.DMA,
        ],
    )
    def k(table_hbm, idx_hbm, out_hbm, idx_v, bufs, gsem, ssem0, ssem1):
        wid = lax.axis_index("s") * 2 + lax.axis_index("c")
        row0 = wid * CPW
        pltpu.sync_copy(idx_hbm.at[wid], idx_v)

        pltpu.async_copy(table_hbm.at[idx_v.at[0]], bufs.at[0], gsem)

        def body(j, _):
            slot = lax.rem(j, 2)
            nxt = 1 - slot

            @pl.when(j + 1 < CPW)
            def _():
                pltpu.async_copy(table_hbm.at[idx_v.at[j + 1]], bufs.at[nxt], gsem)

            pltpu.make_async_copy(table_hbm.at[idx_v.at[j]], bufs.at[slot], gsem).wait()
            dst = out_hbm.at[pl.ds((row0 + j) * CH, CH)]

            @pl.when(slot == 0)
            def _():
                pltpu.async_copy(bufs.at[0], dst, ssem0).wait()

            @pl.when(slot == 1)
            def _():
                pltpu.async_copy(bufs.at[1], dst, ssem1).wait()

            return 0

        lax.fori_loop(0, CPW, body, 0)

    return k(table, idx3d)


def _embed_body(x_ref, w_ref, b_ref, o_ref):
    o_ref[...] = (
        jnp.dot(x_ref[...], w_ref[...], preferred_element_type=jnp.float32)
        + b_ref[...]
    )


def _embed(atom_fea, emb_Wp, emb_bp):
    return pl.pallas_call(
        _embed_body,
        grid=(NB3,),
        in_specs=[
            pl.BlockSpec((BN3, ORIG), lambda i: (i, 0)),
            pl.BlockSpec((ORIG, D2), lambda i: (0, 0)),
            pl.BlockSpec((1, D2), lambda i: (0, 0)),
        ],
        out_specs=pl.BlockSpec((BN3, D2), lambda i: (i, 0)),
        out_shape=jax.ShapeDtypeStruct((N, D2), jnp.float32),
    )(atom_fea, emb_Wp, emb_bp)


def _z_halves(atom_ref, anbr_ref, fea_ref, wsf, wsc, wnf, wnc, wff, wfc):
    a = atom_ref[...]
    nb = anbr_ref[...]
    fe = fea_ref[...].reshape(EB, NBR)
    f32 = jnp.float32
    sf = jnp.dot(a, wsf[...], preferred_element_type=f32)
    sc = jnp.dot(a, wsc[...], preferred_element_type=f32)
    nf = jnp.dot(nb, wnf[...], preferred_element_type=f32)
    nc = jnp.dot(nb, wnc[...], preferred_element_type=f32)
    ff = jnp.dot(fe, wff[...], preferred_element_type=f32)
    fc = jnp.dot(fe, wfc[...], preferred_element_type=f32)
    zf = nf.reshape(BN1, M, D) + ff.reshape(BN1, M, D) + sf[:, None, :]
    zc = nc.reshape(BN1, M, D) + fc.reshape(BN1, M, D) + sc[:, None, :]
    return zf, zc


def _p1_body(atom_ref, anbr_ref, fea_ref, wsf, wsc, wnf, wnc, wff, wfc,
             outf_ref, outc_ref, accf, accc):
    i = pl.program_id(0)

    @pl.when(i == 0)
    def _():
        accf[...] = jnp.zeros_like(accf)
        accc[...] = jnp.zeros_like(accc)

    zf, zc = _z_halves(atom_ref, anbr_ref, fea_ref, wsf, wsc, wnf, wnc, wff, wfc)
    accf[0, :] += jnp.sum(zf, axis=(0, 1))
    accf[1, :] += jnp.sum(zf * zf, axis=(0, 1))
    accc[0, :] += jnp.sum(zc, axis=(0, 1))
    accc[1, :] += jnp.sum(zc * zc, axis=(0, 1))

    @pl.when(i == NB1 - 1)
    def _():
        for acc, out in ((accf, outf_ref), (accc, outc_ref)):
            mu = acc[0, :] / NM
            var = acc[1, :] / NM - mu * mu
            out[0, :] = mu
            out[1, :] = lax.rsqrt(var + EPS)


def _pass1(atom, anbr, fea2, ws):
    return pl.pallas_call(
        _p1_body,
        grid=(NB1,),
        in_specs=[
            pl.BlockSpec((BN1, D2), lambda i: (i, 0)),
            pl.BlockSpec((EB, D2), lambda i: (i, 0)),
            pl.BlockSpec((BN1, M, NBR), lambda i: (i, 0, 0)),
        ] + [pl.BlockSpec(w.shape, lambda i: (0, 0)) for w in ws],
        out_specs=[
            pl.BlockSpec((2, D), lambda i: (0, 0)),
            pl.BlockSpec((2, D), lambda i: (0, 0)),
        ],
        out_shape=[
            jax.ShapeDtypeStruct((2, D), jnp.float32),
            jax.ShapeDtypeStruct((2, D), jnp.float32),
        ],
        scratch_shapes=[
            pltpu.VMEM((2, D), jnp.float32),
            pltpu.VMEM((2, D), jnp.float32),
        ],
    )(atom, anbr, fea2, *ws)


def _p2_body(atom_ref, anbr_ref, fea_ref, wsf, wsc, wnf, wnc, wff, wfc,
             stf, stc, g1f, b1f, g1c, b1c,
             sum_ref, out2_ref, acc2):
    i = pl.program_id(0)

    @pl.when(i == 0)
    def _():
        acc2[...] = jnp.zeros_like(acc2)

    zf, zc = _z_halves(atom_ref, anbr_ref, fea_ref, wsf, wsc, wnf, wnc, wff, wfc)
    af = stf[1, :] * g1f[0, :]
    cf = b1f[0, :] - stf[0, :] * af
    ac = stc[1, :] * g1c[0, :]
    cc = b1c[0, :] - stc[0, :] * ac
    nzf = zf * af[None, None, :] + cf[None, None, :]
    nzc = zc * ac[None, None, :] + cc[None, None, :]
    filt = jax.nn.sigmoid(nzf)
    core = jax.nn.softplus(nzc)
    s = jnp.sum(filt * core, axis=1)
    sum_ref[...] = s
    acc2[0, :] += jnp.sum(s, axis=0)
    acc2[1, :] += jnp.sum(s * s, axis=0)

    @pl.when(i == NB1 - 1)
    def _():
        mu = acc2[0, :] / N
        var = acc2[1, :] / N - mu * mu
        out2_ref[0, :] = mu
        out2_ref[1, :] = lax.rsqrt(var + EPS)


def _pass2(atom, anbr, fea2, ws, stf, stc, g1f, b1f, g1c, b1c):
    return pl.pallas_call(
        _p2_body,
        grid=(NB1,),
        in_specs=[
            pl.BlockSpec((BN1, D2), lambda i: (i, 0)),
            pl.BlockSpec((EB, D2), lambda i: (i, 0)),
            pl.BlockSpec((BN1, M, NBR), lambda i: (i, 0, 0)),
        ] + [pl.BlockSpec(w.shape, lambda i: (0, 0)) for w in ws]
          + [pl.BlockSpec((2, D), lambda i: (0, 0))] * 2
          + [pl.BlockSpec((1, D), lambda i: (0, 0))] * 4,
        out_specs=[
            pl.BlockSpec((BN1, D), lambda i: (i, 0)),
            pl.BlockSpec((2, D), lambda i: (0, 0)),
        ],
        out_shape=[
            jax.ShapeDtypeStruct((N, D), jnp.float32),
            jax.ShapeDtypeStruct((2, D), jnp.float32),
        ],
        scratch_shapes=[pltpu.VMEM((2, D), jnp.float32)],
    )(atom, anbr, fea2, *ws, stf, stc, g1f, b1f, g1c, b1c)


def _p3_body(atom_ref, sum_ref, st2, g2, b2, o_ref):
    s = (sum_ref[...] - st2[0, :][None, :]) * st2[1, :][None, :] \
        * g2[0, :][None, :] + b2[0, :][None, :]
    sp = jnp.concatenate([s, jnp.zeros((BN3, D2 - D), jnp.float32)], axis=1)
    o_ref[...] = jax.nn.softplus(atom_ref[...] + sp)


def _pass3(atom, summed, st2, g2, b2):
    return pl.pallas_call(
        _p3_body,
        grid=(NB3,),
        in_specs=[
            pl.BlockSpec((BN3, D2), lambda i: (i, 0)),
            pl.BlockSpec((BN3, D), lambda i: (i, 0)),
            pl.BlockSpec((2, D), lambda i: (0, 0)),
            pl.BlockSpec((1, D), lambda i: (0, 0)),
            pl.BlockSpec((1, D), lambda i: (0, 0)),
        ],
        out_specs=pl.BlockSpec((BN3, D2), lambda i: (i, 0)),
        out_shape=jax.ShapeDtypeStruct((N, D2), jnp.float32),
    )(atom, summed, st2, g2, b2)


def _pool_body(atom_ref, ids_ref, cfw, cfb, outw, outb, o_ref, acc_s, acc_c):
    i = pl.program_id(0)

    @pl.when(i == 0)
    def _():
        acc_s[...] = jnp.zeros_like(acc_s)
        acc_c[...] = jnp.zeros_like(acc_c)

    ids = ids_ref[0, 0, :]
    oh = (lax.broadcasted_iota(jnp.int32, (NCRYS, BP), 0)
          == ids[None, :]).astype(jnp.float32)
    acc_s[...] += jnp.dot(oh, atom_ref[...], preferred_element_type=jnp.float32)
    acc_c[...] += jnp.sum(oh, axis=1, keepdims=True)

    @pl.when(i == NBP - 1)
    def _():
        mean = acc_s[...] / jnp.maximum(acc_c[...], 1.0)
        t = jax.nn.softplus(mean)
        h = jax.nn.softplus(
            jnp.dot(t, cfw[...], preferred_element_type=jnp.float32) + cfb[...]
        )
        o_ref[...] = jnp.sum(h * outw[...], axis=1, keepdims=True) + outb[...]


def _pool(atom, ids3d, cf_Wp, cf_b, out_W, out_b):
    return pl.pallas_call(
        _pool_body,
        grid=(NBP,),
        in_specs=[
            pl.BlockSpec((BP, D2), lambda i: (i, 0)),
            pl.BlockSpec((1, 1, BP), lambda i: (i, 0, 0)),
            pl.BlockSpec((D2, H), lambda i: (0, 0)),
            pl.BlockSpec((1, H), lambda i: (0, 0)),
            pl.BlockSpec((1, H), lambda i: (0, 0)),
            pl.BlockSpec((1, 1), lambda i: (0, 0)),
        ],
        out_specs=pl.BlockSpec((NCRYS, 1), lambda i: (0, 0)),
        out_shape=jax.ShapeDtypeStruct((NCRYS, 1), jnp.float32),
        scratch_shapes=[
            pltpu.VMEM((NCRYS, D2), jnp.float32),
            pltpu.VMEM((NCRYS, 1), jnp.float32),
        ],
    )(atom, ids3d, cf_Wp, cf_b, out_W, out_b)


def _rpad(a, rows):
    return jnp.pad(a, ((0, rows - a.shape[0]), (0, 0)))


def kernel(atom_fea, nbr_fea, nbr_fea_idx, crystal_ids, emb_W, emb_b,
           fc_W, fc_b, bn1_g, bn1_b, bn2_g, bn2_b, cf_W, cf_b, out_W, out_b):
    idx_flat = nbr_fea_idx.reshape(NM).astype(jnp.int32)
    idx3d = jnp.pad(idx_flat, (0, NMP - NM)).reshape(NW, CPW, CH)
    ids3d = crystal_ids.astype(jnp.int32).reshape(NBP, 1, BP)
    emb_Wp = jnp.pad(emb_W, ((0, 0), (0, D2 - D)))
    emb_bp = jnp.pad(emb_b, (0, D2 - D)).reshape(1, D2)
    cf_Wp = _rpad(cf_W, D2)

    atom = _embed(atom_fea, emb_Wp, emb_bp)

    for i in range(NCONV):
        w = fc_W[i]
        ws = (_rpad(w[:D, :D], D2), _rpad(w[:D, D:], D2),
              _rpad(w[D:2 * D, :D], D2), _rpad(w[D:2 * D, D:], D2),
              w[2 * D:, :D], w[2 * D:, D:])
        g1f = bn1_g[i][:D].reshape(1, D)
        g1c = bn1_g[i][D:].reshape(1, D)
        b1f = bn1_b[i][:D].reshape(1, D)
        b1c = bn1_b[i][D:].reshape(1, D)
        g2 = bn2_g[i].reshape(1, D)
        b2 = bn2_b[i].reshape(1, D)

        anbr = _sc_gather(atom, idx3d)
        stf, stc = _pass1(atom, anbr, nbr_fea, ws)
        summed, st2 = _pass2(atom, anbr, nbr_fea, ws, stf, stc, g1f, b1f, g1c, b1c)
        atom = _pass3(atom, summed, st2, g2, b2)

    return _pool(atom, ids3d, cf_Wp, cf_b.reshape(1, H), out_W.reshape(1, H),
                 out_b.reshape(1, 1))
